# baseline (speedup 1.0000x reference)
import functools

import jax
import jax.numpy as jnp
from jax import lax
from jax.experimental import pallas as pl
from jax.experimental.pallas import tpu as pltpu
from jax.experimental.pallas import tpu_sc as plsc

_NC = 2
_NS = 16
_NW = _NC * _NS
_L = 16
_C = 40
_CP = 48
_D = 256


@functools.lru_cache(maxsize=None)
def _make_kernel(E):
    assert E % (_NW * _C) == 0
    epw = E // _NW
    nt = epw // _C
    mesh = plsc.VectorSubcoreMesh(core_axis_name="c", subcore_axis_name="s")

    @functools.partial(
        pl.kernel,
        out_type=jax.ShapeDtypeStruct((E,), jnp.float32),
        mesh=mesh,
        compiler_params=pltpu.CompilerParams(needs_layout_passes=False),
        scratch_types=[
            pltpu.VMEM((2, _C), jnp.int32),
            pltpu.VMEM((2, _C), jnp.int32),
            pltpu.VMEM((2, _CP, _D), jnp.float32),
            pltpu.VMEM((2, _CP, _D), jnp.float32),
            pltpu.VMEM((epw + _CP - _C,), jnp.float32),
            pltpu.VMEM((_L * _L,), jnp.float32),
            pltpu.SemaphoreType.DMA,
            pltpu.SemaphoreType.DMA,
        ],
    )
    def ker(emb, src, dst, out, sidx, didx, srows, drows, scores, accbuf,
            sem0, sem1):
        wid = lax.axis_index("s") * _NC + lax.axis_index("c")
        base = wid * epw
        sems = (sem0, sem1)

        def fire(g, b):
            off = base + g * _C
            pltpu.sync_copy(src.at[pl.ds(off, _C)], sidx.at[b])
            pltpu.sync_copy(dst.at[pl.ds(off, _C)], didx.at[b])
            pltpu.async_copy(emb.at[sidx.at[b]], srows.at[b, pl.ds(0, _C)],
                             sems[b])
            pltpu.async_copy(emb.at[didx.at[b]], drows.at[b, pl.ds(0, _C)],
                             sems[b])

        def drain(b):
            pltpu.make_async_copy(emb.at[sidx.at[b]],
                                  srows.at[b, pl.ds(0, _C)], sems[b]).wait()
            pltpu.make_async_copy(emb.at[didx.at[b]],
                                  drows.at[b, pl.ds(0, _C)], sems[b]).wait()

        fire(0, 0)

        @pl.loop(0, nt + 1, step=2)
        def _chunks(t):
            for b in range(2):
                g = t + b

                @pl.when(g < nt)
                def _():
                    @pl.when(g + 1 < nt)
                    def _():
                        fire(g + 1, 1 - b)

                    drain(b)
                    for j in range(_CP // _L):
                        for m in range(_L):
                            e = j * _L + m
                            acc = (srows[b, e, pl.ds(0, _L)]
                                   * drows[b, e, pl.ds(0, _L)])
                            for k in range(1, _D // _L):
                                acc = acc + (srows[b, e, pl.ds(k * _L, _L)]
                                             * drows[b, e, pl.ds(k * _L, _L)])
                            accbuf[pl.ds(m * _L, _L)] = acc
                        iot = lax.iota(jnp.int32, _L) * _L
                        svec = plsc.load_gather(accbuf, [iot])
                        for l in range(1, _L):
                            svec = svec + plsc.load_gather(accbuf, [iot + l])
                        scores[pl.ds(g * _C + j * _L, _L)] = svec

        pltpu.sync_copy(scores.at[pl.ds(0, epw)], out.at[pl.ds(base, epw)])

    return ker


def kernel(embedding, edge_index):
    E = edge_index.shape[1]
    ei = edge_index.astype(jnp.int32)
    out = _make_kernel(E)(embedding, ei[0], ei[1])
    return out[:, None]


# speedup vs baseline: 1.6616x; 1.6616x over previous
import functools

import jax
import jax.numpy as jnp
from jax import lax
from jax.experimental import pallas as pl
from jax.experimental.pallas import tpu as pltpu
from jax.experimental.pallas import tpu_sc as plsc

_NC = 2
_NS = 16
_NW = _NC * _NS
_L = 16
_C = 40
_CP = 48
_D = 256


@functools.lru_cache(maxsize=None)
def _make_kernel(E):
    assert E % (_NW * _C) == 0
    epw = E // _NW
    nt = epw // _C
    mesh = plsc.VectorSubcoreMesh(core_axis_name="c", subcore_axis_name="s")

    @functools.partial(
        pl.kernel,
        out_type=jax.ShapeDtypeStruct((E,), jnp.float32),
        mesh=mesh,
        compiler_params=pltpu.CompilerParams(needs_layout_passes=False),
        scratch_types=[
            pltpu.VMEM((epw,), jnp.int32),
            pltpu.VMEM((epw,), jnp.int32),
            pltpu.VMEM((2, _CP, _D), jnp.float32),
            pltpu.VMEM((2, _CP, _D), jnp.float32),
            pltpu.VMEM((epw + _CP - _C,), jnp.float32),
            pltpu.VMEM((_L * _L,), jnp.float32),
            pltpu.SemaphoreType.DMA,
            pltpu.SemaphoreType.DMA,
        ],
    )
    def ker(emb, src, dst, out, sidx, didx, srows, drows, scores, accbuf,
            sem0, sem1):
        wid = lax.axis_index("s") * _NC + lax.axis_index("c")
        base = wid * epw
        sems = (sem0, sem1)

        pltpu.sync_copy(src.at[pl.ds(base, epw)], sidx)
        pltpu.sync_copy(dst.at[pl.ds(base, epw)], didx)

        def fire(g, b):
            pltpu.async_copy(emb.at[sidx.at[pl.ds(g * _C, _C)]],
                             srows.at[b, pl.ds(0, _C)], sems[b])
            pltpu.async_copy(emb.at[didx.at[pl.ds(g * _C, _C)]],
                             drows.at[b, pl.ds(0, _C)], sems[b])

        def drain(g, b):
            pltpu.make_async_copy(emb.at[sidx.at[pl.ds(g * _C, _C)]],
                                  srows.at[b, pl.ds(0, _C)], sems[b]).wait()
            pltpu.make_async_copy(emb.at[didx.at[pl.ds(g * _C, _C)]],
                                  drows.at[b, pl.ds(0, _C)], sems[b]).wait()

        fire(0, 0)

        @pl.loop(0, nt + 1, step=2)
        def _chunks(t):
            for b in range(2):
                g = t + b

                @pl.when(g < nt)
                def _():
                    @pl.when(g + 1 < nt)
                    def _():
                        fire(g + 1, 1 - b)

                    drain(g, b)
                    for j in range(_CP // _L):
                        for m in range(_L if j < _C // _L else _C % _L):
                            e = j * _L + m
                            acc = (srows[b, e, pl.ds(0, _L)]
                                   * drows[b, e, pl.ds(0, _L)])
                            for k in range(1, _D // _L):
                                acc = acc + (srows[b, e, pl.ds(k * _L, _L)]
                                             * drows[b, e, pl.ds(k * _L, _L)])
                            accbuf[pl.ds(m * _L, _L)] = acc
                        iot = lax.iota(jnp.int32, _L) * _L
                        svec = plsc.load_gather(accbuf, [iot])
                        for l in range(1, _L):
                            svec = svec + plsc.load_gather(accbuf, [iot + l])
                        scores[pl.ds(g * _C + j * _L, _L)] = svec

        pltpu.sync_copy(scores.at[pl.ds(0, epw)], out.at[pl.ds(base, epw)])

    return ker


def kernel(embedding, edge_index):
    E = edge_index.shape[1]
    ei = edge_index.astype(jnp.int32)
    out = _make_kernel(E)(embedding, ei[0], ei[1])
    return out[:, None]
